# baseline (speedup 1.0000x reference)
import functools

import jax
import jax.numpy as jnp
from jax import lax
from jax.experimental import pallas as pl
from jax.experimental.pallas import tpu as pltpu
from jax.experimental.pallas import tpu_sc as plsc

_NUM_BINS = 15
_MIN = -3.0
_MAX = 3.0
_D = 64
_B = 16384
_NC = 2
_NS = 16
_NW = _NC * _NS
_BPW = _B // _NW
_L = 16

_SCALE = (_NUM_BINS - 1) / (_MAX - _MIN)
_MAGIC = 2.0 ** 23


def _sc_embed(r, table):
    mesh = plsc.VectorSubcoreMesh(core_axis_name="c", subcore_axis_name="s")

    @functools.partial(
        pl.kernel,
        mesh=mesh,
        out_type=jax.ShapeDtypeStruct((_B, _D), jnp.float32),
        compiler_params=pltpu.CompilerParams(use_tc_tiling_on_sc=False),
        scratch_types=[
            pltpu.VMEM((_BPW,), jnp.float32),
            pltpu.VMEM((_BPW,), jnp.int32),
            pltpu.VMEM((_BPW, _D), jnp.float32),
            pltpu.SemaphoreType.DMA,
        ],
    )
    def k(r_hbm, table_hbm, out_hbm, r_v, idx_v, rows_v, sem):
        wid = lax.axis_index("s") * _NC + lax.axis_index("c")
        base = wid * _BPW
        pltpu.sync_copy(r_hbm.at[pl.ds(base, _BPW)], r_v)
        for i in range(_BPW // _L):
            rv = r_v[pl.ds(i * _L, _L)]
            t = jnp.minimum(jnp.maximum(rv, _MIN), _MAX)
            x = (t - _MIN) * jnp.float32(_SCALE)
            f = (x + _MAGIC) - _MAGIC
            idx_v[pl.ds(i * _L, _L)] = f.astype(jnp.int32)
        pltpu.async_copy(table_hbm.at[idx_v], rows_v, sem).wait()
        pltpu.sync_copy(rows_v, out_hbm.at[pl.ds(base, _BPW)])

    return k(r, table)


def kernel(r, table):
    return _sc_embed(r, table)


# speedup vs baseline: 3.7389x; 3.7389x over previous
import functools

import jax
import jax.numpy as jnp
from jax import lax
from jax.experimental import pallas as pl
from jax.experimental.pallas import tpu as pltpu
from jax.experimental.pallas import tpu_sc as plsc

_NUM_BINS = 15
_MIN = -3.0
_MAX = 3.0
_D = 64
_B = 16384
_NC = 2
_NS = 16
_NW = _NC * _NS
_BPW = _B // _NW
_L = 16

_SCALE = (_NUM_BINS - 1) / (_MAX - _MIN)
_MAGIC = 2.0 ** 23


def _sc_embed(r, table):
    mesh = plsc.VectorSubcoreMesh(core_axis_name="c", subcore_axis_name="s")

    @functools.partial(
        pl.kernel,
        mesh=mesh,
        out_type=jax.ShapeDtypeStruct((_B, _D), jnp.float32),
        compiler_params=pltpu.CompilerParams(use_tc_tiling_on_sc=False),
        scratch_types=[
            pltpu.VMEM((_BPW,), jnp.float32),
            pltpu.VMEM((_BPW,), jnp.int32),
            pltpu.VMEM((_BPW, _D), jnp.float32),
            pltpu.VMEM_SHARED((_NUM_BINS, _D), jnp.float32),
            pltpu.SemaphoreType.DMA,
        ],
    )
    def k(r_hbm, table_hbm, out_hbm, r_v, idx_v, rows_v, table_s, sem):
        sid = lax.axis_index("s")
        wid = sid * _NC + lax.axis_index("c")
        base = wid * _BPW
        @pl.when(sid == 0)
        def _stage_table():
            pltpu.sync_copy(table_hbm, table_s)
        pltpu.sync_copy(r_hbm.at[pl.ds(base, _BPW)], r_v)
        for i in range(_BPW // _L):
            rv = r_v[pl.ds(i * _L, _L)]
            t = jnp.minimum(jnp.maximum(rv, _MIN), _MAX)
            x = (t - _MIN) * jnp.float32(_SCALE)
            f = (x + _MAGIC) - _MAGIC
            idx_v[pl.ds(i * _L, _L)] = f.astype(jnp.int32)
        plsc.subcore_barrier()
        pltpu.async_copy(table_s.at[idx_v], rows_v, sem).wait()
        pltpu.sync_copy(rows_v, out_hbm.at[pl.ds(base, _BPW)])

    return k(r, table)


def kernel(r, table):
    return _sc_embed(r, table)
